# Initial kernel scaffold; baseline (speedup 1.0000x reference)
import functools

import jax
import jax.numpy as jnp
from jax import lax
from jax.experimental import pallas as pl
from jax.experimental.pallas import tpu as pltpu
from jax.experimental.pallas import tpu_sc as plsc

B = 16384
DIM = 32
NBUCKETS = 1000
OUT_D = 5 * DIM + 2
MEAN = 0.5
VAR = 1.0 / 12.0
INV_STD = 1.0 / (VAR + 1e-6) ** 0.5

_info = plsc.get_sparse_core_info()
NC, NS, L = _info.num_cores, _info.num_subcores, _info.num_lanes
NW = NC * NS
ROWS_PER_W = B // NW
CHUNK = 128
NCHUNK = ROWS_PER_W // CHUNK
NGROUP = CHUNK // 16
NSTEP = 10


def _searchsorted_chunk(bucket_ref, val_ref, idx_ref):
    def group(g, _):
        v = val_ref[pl.ds(g * 16, 16)]
        lo = jnp.zeros((16,), jnp.int32)
        hi = jnp.full((16,), NBUCKETS, jnp.int32)

        def step(_, carry):
            lo, hi = carry
            mid = (lo + hi) >> 1
            a = plsc.load_gather(bucket_ref, [mid])
            p = a < v
            return jnp.where(p, mid + 1, lo), jnp.where(p, hi, mid)

        lo, hi = lax.fori_loop(0, NSTEP, step, (lo, hi))
        idx_ref[pl.ds(g * 16, 16)] = lo
        return 0

    lax.fori_loop(0, NGROUP, group, 0)


def _scatter_column(tile_ref, val_ref, col):
    def group(g, _):
        v = (val_ref[pl.ds(g * 16, 16)] - MEAN) * INV_STD
        rows = g * 16 + lax.iota(jnp.int32, 16)
        cols = jnp.full((16,), col, jnp.int32)
        plsc.store_scatter(tile_ref, [rows, cols], v)
        return 0

    lax.fori_loop(0, NGROUP, group, 0)


def _encode(uid_hbm, gid_hbm, oid_hbm, rat_hbm, ts_hbm,
            utab_hbm, gtab_hbm, otab_hbm, rtab_hbm, ttab_hbm,
            rb_hbm, tb_hbm, out_hbm,
            rb_v, tb_v, uid_v, gid_v, oid_v, rid_v, tid_v,
            rat_v, ts_v, tile_v, sem):
    wid = lax.axis_index("s") * NC + lax.axis_index("c")
    pltpu.sync_copy(rb_hbm, rb_v)
    pltpu.sync_copy(tb_hbm, tb_v)

    for j in range(NCHUNK):
        base = wid * ROWS_PER_W + j * CHUNK
        pltpu.sync_copy(uid_hbm.at[pl.ds(base, CHUNK)], uid_v)
        pltpu.sync_copy(gid_hbm.at[pl.ds(base, CHUNK)], gid_v)
        pltpu.sync_copy(oid_hbm.at[pl.ds(base, CHUNK)], oid_v)
        pltpu.sync_copy(rat_hbm.at[pl.ds(base, CHUNK)], rat_v)
        pltpu.sync_copy(ts_hbm.at[pl.ds(base, CHUNK)], ts_v)

        d1 = pltpu.async_copy(utab_hbm.at[uid_v], tile_v.at[:, pl.ds(0, DIM)], sem)
        d2 = pltpu.async_copy(gtab_hbm.at[gid_v], tile_v.at[:, pl.ds(DIM, DIM)], sem)
        d3 = pltpu.async_copy(otab_hbm.at[oid_v], tile_v.at[:, pl.ds(2 * DIM, DIM)], sem)

        _searchsorted_chunk(rb_v, rat_v, rid_v)
        _searchsorted_chunk(tb_v, ts_v, tid_v)

        d4 = pltpu.async_copy(rtab_hbm.at[rid_v], tile_v.at[:, pl.ds(3 * DIM, DIM)], sem)
        d5 = pltpu.async_copy(ttab_hbm.at[tid_v], tile_v.at[:, pl.ds(4 * DIM + 1, DIM)], sem)

        _scatter_column(tile_v, rat_v, 4 * DIM)
        _scatter_column(tile_v, ts_v, 5 * DIM + 1)

        d1.wait()
        d2.wait()
        d3.wait()
        d4.wait()
        d5.wait()

        pltpu.sync_copy(tile_v, out_hbm.at[pl.ds(base, CHUNK), :])


@jax.jit
def kernel(user_id, user_gender, user_occupation_label, user_rating, timestamp,
           user_table, gender_table, occupation_table, rating_table, timestamp_table,
           rating_buckets, timestamp_buckets):
    run = functools.partial(
        pl.kernel,
        out_type=jax.ShapeDtypeStruct((B, OUT_D), jnp.float32),
        mesh=plsc.VectorSubcoreMesh(core_axis_name="c", subcore_axis_name="s"),
        scratch_types=[
            pltpu.VMEM((NBUCKETS,), jnp.float32),
            pltpu.VMEM((NBUCKETS,), jnp.float32),
            pltpu.VMEM((CHUNK,), jnp.int32),
            pltpu.VMEM((CHUNK,), jnp.int32),
            pltpu.VMEM((CHUNK,), jnp.int32),
            pltpu.VMEM((CHUNK,), jnp.int32),
            pltpu.VMEM((CHUNK,), jnp.int32),
            pltpu.VMEM((CHUNK,), jnp.float32),
            pltpu.VMEM((CHUNK,), jnp.float32),
            pltpu.VMEM((CHUNK, OUT_D), jnp.float32),
            pltpu.SemaphoreType.DMA,
        ],
    )(_encode)
    return run(user_id.astype(jnp.int32), user_gender.astype(jnp.int32),
               user_occupation_label.astype(jnp.int32), user_rating, timestamp,
               user_table, gender_table, occupation_table, rating_table,
               timestamp_table, rating_buckets, timestamp_buckets)


# speedup vs baseline: 10.9175x; 10.9175x over previous
import functools

import jax
import jax.numpy as jnp
from jax import lax
from jax.experimental import pallas as pl
from jax.experimental.pallas import tpu as pltpu
from jax.experimental.pallas import tpu_sc as plsc

B = 16384
DIM = 32
OCC_VOCAB = 22
NBUCKETS = 1000
TAB_ROWS = NBUCKETS + 1
TAB_WORDS = TAB_ROWS * DIM
TAB_PAD = 32128
NB_PAD = 1024
OUT_D = 5 * DIM + 2
MEAN = 0.5
VAR = 1.0 / 12.0
INV_STD = 1.0 / (VAR + 1e-6) ** 0.5

_info = plsc.get_sparse_core_info()
NC, NS, L = _info.num_cores, _info.num_subcores, _info.num_lanes
NW = NC * NS
ROWS_PER_W = B // NW
CHUNK = 128
NCHUNK = ROWS_PER_W // CHUNK
NGROUP = CHUNK // 16
NSTEP = 10


def _gather_sc(uid_hbm, rat_hbm, ts_hbm, utab4_hbm, rtab_hbm, ttab_hbm,
               rb_hbm, tb_hbm, ug_hbm, rg_hbm, tg_hbm,
               rtab_sp, ttab_sp,
               rtab_v, ttab_v, rb_v, tb_v, uid_v, uid4_v, rat_v, ts_v,
               u4_rows, ue_v, re_v, te_v, sem):
    cid = lax.axis_index("c")
    sid = lax.axis_index("s")
    wid = sid * NC + cid

    @pl.when(sid == 0)
    def _():
        pltpu.sync_copy(rtab_hbm, rtab_sp)
        pltpu.sync_copy(ttab_hbm, ttab_sp)

    plsc.subcore_barrier()
    pltpu.sync_copy(rtab_sp, rtab_v)
    pltpu.sync_copy(ttab_sp, ttab_v)
    pltpu.sync_copy(rb_hbm, rb_v)
    pltpu.sync_copy(tb_hbm, tb_v)

    iota16 = lax.iota(jnp.int32, 16)

    for j in range(NCHUNK):
        base = wid * ROWS_PER_W + j * CHUNK
        rows = pl.ds(base, CHUNK)
        pltpu.sync_copy(uid_hbm.at[rows], uid_v)
        pltpu.sync_copy(rat_hbm.at[rows], rat_v)
        pltpu.sync_copy(ts_hbm.at[rows], ts_v)

        def quarter(g, _):
            uidg = uid_v[pl.ds(g * 16, 16)]
            uid4_v[pl.ds(g * 16, 16)] = uidg >> 2
            return 0

        lax.fori_loop(0, NGROUP, quarter, 0)
        du = pltpu.async_copy(utab4_hbm.at[uid4_v], u4_rows, sem)

        def bucketize(g, _):
            gs = pl.ds(g * 16, 16)
            vr = rat_v[gs]
            vt = ts_v[gs]

            def search(bucket_ref, v):
                lo = jnp.zeros((16,), jnp.int32)
                hi = jnp.full((16,), NBUCKETS, jnp.int32)

                def step(_, carry):
                    lo, hi = carry
                    mid = (lo + hi) >> 1
                    p = plsc.load_gather(bucket_ref, [mid]) < v
                    return jnp.where(p, mid + 1, lo), jnp.where(p, hi, mid)

                return lax.fori_loop(0, NSTEP, step, (lo, hi))[0]

            ridx = search(rb_v, vr) * DIM
            tidx = search(tb_v, vt) * DIM
            dst = g * (16 * DIM) + iota16 * DIM
            for c in range(DIM):
                rv = plsc.load_gather(rtab_v, [ridx + c])
                tv = plsc.load_gather(ttab_v, [tidx + c])
                plsc.store_scatter(re_v, [dst + c], rv)
                plsc.store_scatter(te_v, [dst + c], tv)
            return 0

        lax.fori_loop(0, NGROUP, bucketize, 0)

        dr = pltpu.sync_copy(re_v, rg_hbm.at[pl.ds(base * DIM, CHUNK * DIM)])
        dt = pltpu.sync_copy(te_v, tg_hbm.at[pl.ds(base * DIM, CHUNK * DIM)])
        du.wait()

        def extract(g, _):
            uidg = uid_v[pl.ds(g * 16, 16)]
            rows16 = g * 16 + iota16
            sub = (uidg & 3) * DIM
            dst = g * (16 * DIM) + iota16 * DIM
            for c in range(DIM):
                uv = plsc.load_gather(u4_rows, [rows16, sub + c])
                plsc.store_scatter(ue_v, [dst + c], uv)
            return 0

        lax.fori_loop(0, NGROUP, extract, 0)
        pltpu.sync_copy(ue_v, ug_hbm.at[pl.ds(base * DIM, CHUNK * DIM)])


def _concat_tc(u_ref, r_ref, t_ref, gid_ref, oid_ref, rat_ref, ts_ref,
               gtab_ref, otab_ref, out_ref):
    br = u_ref.shape[0]
    g_oh = (gid_ref[...] == lax.broadcasted_iota(jnp.int32, (br, 2), 1)
            ).astype(jnp.float32)
    g_rows = jnp.dot(g_oh, gtab_ref[...], preferred_element_type=jnp.float32,
                     precision=lax.Precision.HIGHEST)
    o_oh = (oid_ref[...] == lax.broadcasted_iota(jnp.int32, (br, OCC_VOCAB), 1)
            ).astype(jnp.float32)
    o_rows = jnp.dot(o_oh, otab_ref[...], preferred_element_type=jnp.float32,
                     precision=lax.Precision.HIGHEST)
    nr = (rat_ref[...] - MEAN) * INV_STD
    nt = (ts_ref[...] - MEAN) * INV_STD
    out_ref[...] = jnp.concatenate(
        [u_ref[...], g_rows, o_rows, r_ref[...], nr, t_ref[...], nt], axis=1)


@jax.jit
def kernel(user_id, user_gender, user_occupation_label, user_rating, timestamp,
           user_table, gender_table, occupation_table, rating_table, timestamp_table,
           rating_buckets, timestamp_buckets):
    user_id = user_id.astype(jnp.int32)
    user_gender = user_gender.astype(jnp.int32)
    user_occupation_label = user_occupation_label.astype(jnp.int32)

    sc_gather = functools.partial(
        pl.kernel,
        out_type=[jax.ShapeDtypeStruct((B * DIM,), jnp.float32)] * 3,
        mesh=plsc.VectorSubcoreMesh(core_axis_name="c", subcore_axis_name="s"),
        compiler_params=pltpu.CompilerParams(needs_layout_passes=False),
        scratch_types=[
            pltpu.VMEM_SHARED((TAB_PAD,), jnp.float32),
            pltpu.VMEM_SHARED((TAB_PAD,), jnp.float32),
            pltpu.VMEM((TAB_PAD,), jnp.float32),
            pltpu.VMEM((TAB_PAD,), jnp.float32),
            pltpu.VMEM((NB_PAD,), jnp.float32),
            pltpu.VMEM((NB_PAD,), jnp.float32),
            pltpu.VMEM((CHUNK,), jnp.int32),
            pltpu.VMEM((CHUNK,), jnp.int32),
            pltpu.VMEM((CHUNK,), jnp.float32),
            pltpu.VMEM((CHUNK,), jnp.float32),
            pltpu.VMEM((CHUNK, 4 * DIM), jnp.float32),
            pltpu.VMEM((CHUNK * DIM,), jnp.float32),
            pltpu.VMEM((CHUNK * DIM,), jnp.float32),
            pltpu.VMEM((CHUNK * DIM,), jnp.float32),
            pltpu.SemaphoreType.DMA,
        ],
    )(_gather_sc)
    ug, rg, tg = sc_gather(
        user_id, user_rating, timestamp,
        user_table.reshape(-1, 4 * DIM),
        jnp.pad(rating_table.reshape(TAB_WORDS), (0, TAB_PAD - TAB_WORDS)),
        jnp.pad(timestamp_table.reshape(TAB_WORDS), (0, TAB_PAD - TAB_WORDS)),
        jnp.pad(rating_buckets, (0, NB_PAD - NBUCKETS), constant_values=jnp.inf),
        jnp.pad(timestamp_buckets, (0, NB_PAD - NBUCKETS), constant_values=jnp.inf))

    br = 1024
    out = pl.pallas_call(
        _concat_tc,
        out_shape=jax.ShapeDtypeStruct((B, OUT_D), jnp.float32),
        grid=(B // br,),
        in_specs=[
            pl.BlockSpec((br, DIM), lambda i: (i, 0)),
            pl.BlockSpec((br, DIM), lambda i: (i, 0)),
            pl.BlockSpec((br, DIM), lambda i: (i, 0)),
            pl.BlockSpec((br, 1), lambda i: (i, 0)),
            pl.BlockSpec((br, 1), lambda i: (i, 0)),
            pl.BlockSpec((br, 1), lambda i: (i, 0)),
            pl.BlockSpec((br, 1), lambda i: (i, 0)),
            pl.BlockSpec((2, DIM), lambda i: (0, 0)),
            pl.BlockSpec((OCC_VOCAB, DIM), lambda i: (0, 0)),
        ],
        out_specs=pl.BlockSpec((br, OUT_D), lambda i: (i, 0)),
    )(ug.reshape(B, DIM), rg.reshape(B, DIM), tg.reshape(B, DIM),
      user_gender.reshape(B, 1), user_occupation_label.reshape(B, 1),
      user_rating.reshape(B, 1), timestamp.reshape(B, 1),
      gender_table, occupation_table)
    return out
